# baseline (speedup 1.0000x reference)
import functools

import jax
import jax.numpy as jnp
from jax import lax
from jax.experimental import pallas as pl
from jax.experimental.pallas import tpu as pltpu
from jax.experimental.pallas import tpu_sc as plsc

T = 2048
D = 768
E = 8
K = 2
F = 768
TK = T * K

BT = 256
NT = T // BT

BG = 256
MAXT = TK // BG + (E - 1)
SLOTS = MAXT * BG

NC = 2
NS = 16
NW = NC * NS
CHUNK = TK // NW
BLK = TK // NS
TPW = T // NW


def _silu(x):
    return x * jax.nn.sigmoid(x)


def _take16(vec, idx):
    return lax.gather(
        vec, idx[:, None],
        dimension_numbers=lax.GatherDimensionNumbers(
            offset_dims=(), collapsed_slice_dims=(0,), start_index_map=(0,)),
        slice_sizes=(1,),
        mode=lax.GatherScatterMode.PROMISE_IN_BOUNDS)


def _router_body(x_ref, gwt_ref, eidx_ref, wpair_ref):
    x = x_ref[...]
    logits = jnp.dot(x, gwt_ref[...], preferred_element_type=jnp.float32)
    p = jax.nn.sigmoid(logits)
    iota = lax.broadcasted_iota(jnp.int32, p.shape, 1)
    v1 = jnp.max(p, axis=-1, keepdims=True)
    i1 = jnp.min(jnp.where(p == v1, iota, E), axis=-1, keepdims=True)
    p2 = jnp.where(iota == i1, -jnp.inf, p)
    v2 = jnp.max(p2, axis=-1, keepdims=True)
    i2 = jnp.min(jnp.where(p2 == v2, iota, E), axis=-1, keepdims=True)
    denom = v1 + v2
    eidx_ref[...] = jnp.concatenate([i1, i2], axis=-1)
    wpair_ref[...] = jnp.concatenate([v1 / denom, v2 / denom], axis=-1)


def _router(x, gwt):
    return pl.pallas_call(
        _router_body,
        grid=(NT,),
        in_specs=[
            pl.BlockSpec((BT, D), lambda t: (t, 0)),
            pl.BlockSpec((D, E), lambda t: (0, 0)),
        ],
        out_specs=[
            pl.BlockSpec((BT, K), lambda t: (t, 0)),
            pl.BlockSpec((BT, K), lambda t: (t, 0)),
        ],
        out_shape=[
            jax.ShapeDtypeStruct((T, K), jnp.int32),
            jax.ShapeDtypeStruct((T, K), jnp.float32),
        ],
    )(x, gwt)


def _dispatch_body(eidx_hbm, x_hbm,
                   xs_hbm, pos_hbm, te_hbm,
                   eid_v, hstage_v, hist_v, pos_v, posmat_v, idxg_v,
                   xbuf_v, te_v, hist_sh, sem_g, sem_s):
    c = lax.axis_index("c")
    s = lax.axis_index("s")
    wid = s * NC + c
    lane = lax.broadcasted_iota(jnp.int32, (16,), 0)
    z16 = jnp.zeros((16,), jnp.int32)

    pltpu.sync_copy(eidx_hbm.at[pl.ds(s * BLK, BLK)], eid_v)

    def h_body(i, carry):
        tot, half = carry
        v = eid_v[pl.ds(i * 16, 16)]
        contrib = z16
        def e_body(e, acc):
            m = v == e
            csum = plsc.cumsum(m.astype(jnp.int32))
            cnt = _take16(csum, jnp.full((16,), 15, jnp.int32))
            return acc + jnp.where(lane == e, cnt, 0)
        contrib = lax.fori_loop(0, E, e_body, contrib)
        flag = (i < 8).astype(jnp.int32)
        return tot + contrib, half + contrib * flag

    blk_hist, half_hist = lax.fori_loop(0, 16, h_body, (z16, z16))
    hstage_v[...] = blk_hist
    pltpu.sync_copy(hstage_v, hist_sh.at[pl.ds(s * 16, 16)])
    plsc.subcore_barrier()
    pltpu.sync_copy(hist_sh, hist_v)

    def r_body(jj, carry):
        tot, pre = carry
        row = hist_v[pl.ds(jj * 16, 16)]
        return tot + row, pre + row * (jj < s).astype(jnp.int32)

    totals, pre_blk = lax.fori_loop(0, NS, r_body, (z16, z16))

    tiles_e = (totals + (BG - 1)) >> 8
    ts_inc = plsc.cumsum(tiles_e)
    tile_start = ts_inc - tiles_e
    slot_base = tile_start * BG
    bases = slot_base + pre_blk + half_hist * c

    for jv in range(2):
        jvec = lane + 16 * jv
        acc = jnp.full((16,), -1, jnp.int32)
        for e in range(E):
            st = _take16(tile_start, jnp.full((16,), e, jnp.int32))
            acc = acc + (jvec >= st).astype(jnp.int32)
        te_v[pl.ds(16 * jv, 16)] = acc

    @pl.when(wid == 0)
    def _():
        pltpu.sync_copy(te_v, te_hbm)

    def j_body(j, run):
        v = eid_v[pl.ds(c * CHUNK + 16 * j, 16)]
        rank = z16
        cnt = z16
        for e in range(E):
            m = v == e
            csum = plsc.cumsum(m.astype(jnp.int32))
            rank = jnp.where(m, csum, rank)
            cnt = cnt + jnp.where(lane == e,
                                  _take16(csum, jnp.full((16,), 15, jnp.int32)), 0)
        basev = _take16(bases + run, v)
        pos16 = basev + rank - 1
        pos_v[pl.ds(16 * j, 16)] = pos16
        posmat_v[j, :] = pos16
        gpair = s * BLK + c * CHUNK + 16 * j
        tok16 = (lane + gpair) >> 1
        idxg_v[...] = tok16
        pltpu.async_copy(x_hbm.at[idxg_v], xbuf_v, sem_g).wait()
        pltpu.async_copy(xbuf_v, xs_hbm.at[posmat_v.at[j]], sem_s).wait()
        return run + cnt

    lax.fori_loop(0, CHUNK // 16, j_body, z16)
    pltpu.sync_copy(pos_v, pos_hbm.at[pl.ds(s * BLK + c * CHUNK, CHUNK)])


def _dispatch(eidx_flat, x):
    mesh = plsc.VectorSubcoreMesh(core_axis_name="c", subcore_axis_name="s", num_cores=NC, num_subcores=NS)
    f = pl.kernel(
        _dispatch_body,
        out_type=[
            jax.ShapeDtypeStruct((SLOTS, D), jnp.float32),
            jax.ShapeDtypeStruct((TK,), jnp.int32),
            jax.ShapeDtypeStruct((32,), jnp.int32),
        ],
        mesh=mesh,
        compiler_params=pltpu.CompilerParams(needs_layout_passes=False),
        scratch_types=[
            pltpu.VMEM((BLK,), jnp.int32),
            pltpu.VMEM((16,), jnp.int32),
            pltpu.VMEM((NS * 16,), jnp.int32),
            pltpu.VMEM((CHUNK,), jnp.int32),
            pltpu.VMEM((CHUNK // 16, 16), jnp.int32),
            pltpu.VMEM((16,), jnp.int32),
            pltpu.VMEM((16, D), jnp.float32),
            pltpu.VMEM((32,), jnp.int32),
            pltpu.VMEM_SHARED((NS * 16,), jnp.int32),
            pltpu.SemaphoreType.DMA,
            pltpu.SemaphoreType.DMA,
        ],
    )
    return f(eidx_flat, x)


def _group_body(te_ref, xs_ref, w1_ref, w2_ref, ys_ref):
    x = xs_ref[...].astype(jnp.bfloat16)
    h = jnp.dot(x, w1_ref[0], preferred_element_type=jnp.float32)
    a = (_silu(h[:, :F]) * h[:, F:]).astype(jnp.bfloat16)
    ys_ref[...] = jnp.dot(a, w2_ref[0], preferred_element_type=jnp.float32)


def _grouped(tile_expert, x_slots, w1, w2):
    grid_spec = pltpu.PrefetchScalarGridSpec(
        num_scalar_prefetch=1,
        grid=(MAXT,),
        in_specs=[
            pl.BlockSpec((BG, D), lambda t, te: (t, 0)),
            pl.BlockSpec((1, D, 2 * F), lambda t, te: (te[t], 0, 0)),
            pl.BlockSpec((1, F, D), lambda t, te: (te[t], 0, 0)),
        ],
        out_specs=pl.BlockSpec((BG, D), lambda t, te: (t, 0)),
    )
    return pl.pallas_call(
        _group_body,
        grid_spec=grid_spec,
        out_shape=jax.ShapeDtypeStruct((SLOTS, D), jnp.float32),
    )(tile_expert, x_slots, w1, w2)


def _combine_body(ys_hbm, pos_hbm, wpair_hbm, routed_hbm, pos_v, wv_v,
                  posmat_v, ybuf_v, obuf_v, sem):
    c = lax.axis_index("c")
    s = lax.axis_index("s")
    wid = s * NC + c
    pltpu.sync_copy(pos_hbm.at[pl.ds(wid * (2 * TPW), 2 * TPW)], pos_v)
    pltpu.sync_copy(wpair_hbm.at[pl.ds(wid * (2 * TPW), 2 * TPW)], wv_v)

    def j_body(j, _):
        posmat_v[0, :] = pos_v[pl.ds(16 * j, 16)]
        pltpu.async_copy(ys_hbm.at[posmat_v.at[0]], ybuf_v, sem).wait()
        w16 = wv_v[pl.ds(16 * j, 16)]
        wbc = [_take16(w16, jnp.full((16,), r, jnp.int32)) for r in range(16)]

        def k_body(k, _2):
            for i in range(8):
                a = ybuf_v[2 * i, pl.ds(16 * k, 16)]
                b = ybuf_v[2 * i + 1, pl.ds(16 * k, 16)]
                obuf_v[i, pl.ds(16 * k, 16)] = a * wbc[2 * i] + b * wbc[2 * i + 1]
            return 0

        lax.fori_loop(0, D // 16, k_body, 0)
        pltpu.sync_copy(obuf_v, routed_hbm.at[pl.ds(wid * TPW + 8 * j, 8)])
        return 0

    lax.fori_loop(0, TPW // 8, j_body, 0)


def _combine(y_slots, pos_pair, wpair_flat):
    mesh = plsc.VectorSubcoreMesh(core_axis_name="c", subcore_axis_name="s", num_cores=NC, num_subcores=NS)
    f = pl.kernel(
        _combine_body,
        out_type=jax.ShapeDtypeStruct((T, D), jnp.float32),
        mesh=mesh,
        compiler_params=pltpu.CompilerParams(needs_layout_passes=False),
        scratch_types=[
            pltpu.VMEM((2 * TPW,), jnp.int32),
            pltpu.VMEM((2 * TPW,), jnp.float32),
            pltpu.VMEM((1, 16), jnp.int32),
            pltpu.VMEM((16, D), jnp.float32),
            pltpu.VMEM((8, D), jnp.float32),
            pltpu.SemaphoreType.DMA,
        ],
    )
    return f(y_slots, pos_pair, wpair_flat)


def _shared_body(x_ref, ws1_ref, ws2_ref, out_ref):
    x = x_ref[...].astype(jnp.bfloat16)
    h = jnp.dot(x, ws1_ref[...], preferred_element_type=jnp.float32)
    a = (_silu(h[:, :F]) * h[:, F:]).astype(jnp.bfloat16)
    out_ref[...] = jnp.dot(a, ws2_ref[...], preferred_element_type=jnp.float32)


def _shared(x, ws1, ws2):
    return pl.pallas_call(
        _shared_body,
        grid=(NT,),
        in_specs=[
            pl.BlockSpec((BT, D), lambda t: (t, 0)),
            pl.BlockSpec((D, 2 * F), lambda t: (0, 0)),
            pl.BlockSpec((F, D), lambda t: (0, 0)),
        ],
        out_specs=pl.BlockSpec((BT, D), lambda t: (t, 0)),
        out_shape=jax.ShapeDtypeStruct((T, D), jnp.float32),
    )(x, ws1, ws2)


def kernel(hidden_states, gate_w, w1, w2, ws1, ws2):
    orig_shape = hidden_states.shape
    x = hidden_states.reshape(-1, D)
    eidx, wpair = _router(x, gate_w.T)
    x_slots, pos_pair, tile_expert = _dispatch(eidx.reshape(TK), x)
    shared = _shared(x, ws1.astype(jnp.bfloat16), ws2.astype(jnp.bfloat16))
    y_slots = _grouped(tile_expert, x_slots, w1.astype(jnp.bfloat16),
                       w2.astype(jnp.bfloat16))
    routed = _combine(y_slots, pos_pair, wpair.reshape(TK))
    return shared, routed.reshape(orig_shape)


# speedup vs baseline: 1.2457x; 1.2457x over previous
import functools

import jax
import jax.numpy as jnp
from jax import lax
from jax.experimental import pallas as pl
from jax.experimental.pallas import tpu as pltpu
from jax.experimental.pallas import tpu_sc as plsc

T = 2048
D = 768
E = 8
K = 2
F = 768
TK = T * K

BT = 256
NT = T // BT

BG = 256
MAXT = TK // BG + (E - 1)
SLOTS = MAXT * BG

NC = 2
NS = 16
NW = NC * NS
CHUNK = TK // NW
BLK = TK // NS
TPW = T // NW


def _silu(x):
    return x * jax.nn.sigmoid(x)


def _take16(vec, idx):
    return lax.gather(
        vec, idx[:, None],
        dimension_numbers=lax.GatherDimensionNumbers(
            offset_dims=(), collapsed_slice_dims=(0,), start_index_map=(0,)),
        slice_sizes=(1,),
        mode=lax.GatherScatterMode.PROMISE_IN_BOUNDS)


def _router_body(x_ref, gwt_ref, eidx_ref, wpair_ref):
    x = x_ref[...]
    logits = jnp.dot(x, gwt_ref[...], preferred_element_type=jnp.float32)
    p = jax.nn.sigmoid(logits)
    iota = lax.broadcasted_iota(jnp.int32, p.shape, 1)
    v1 = jnp.max(p, axis=-1, keepdims=True)
    i1 = jnp.min(jnp.where(p == v1, iota, E), axis=-1, keepdims=True)
    p2 = jnp.where(iota == i1, -jnp.inf, p)
    v2 = jnp.max(p2, axis=-1, keepdims=True)
    i2 = jnp.min(jnp.where(p2 == v2, iota, E), axis=-1, keepdims=True)
    denom = v1 + v2
    eidx_ref[...] = jnp.concatenate([i1, i2], axis=-1)
    wpair_ref[...] = jnp.concatenate([v1 / denom, v2 / denom], axis=-1)


def _router(x, gwt):
    return pl.pallas_call(
        _router_body,
        grid=(NT,),
        in_specs=[
            pl.BlockSpec((BT, D), lambda t: (t, 0)),
            pl.BlockSpec((D, E), lambda t: (0, 0)),
        ],
        out_specs=[
            pl.BlockSpec((BT, K), lambda t: (t, 0)),
            pl.BlockSpec((BT, K), lambda t: (t, 0)),
        ],
        out_shape=[
            jax.ShapeDtypeStruct((T, K), jnp.int32),
            jax.ShapeDtypeStruct((T, K), jnp.float32),
        ],
    )(x, gwt)


def _dispatch_body(eidx_hbm, x_hbm,
                   xs_hbm, pos_hbm, te_hbm,
                   eid_v, hstage_v, hist_v, pos_v, posw_v, idxg_v,
                   xbuf_v, te_v, hist_sh, sem_g, sem_s):
    c = lax.axis_index("c")
    s = lax.axis_index("s")
    wid = s * NC + c
    lane = lax.broadcasted_iota(jnp.int32, (16,), 0)
    z16 = jnp.zeros((16,), jnp.int32)
    gbase = s * BLK + c * CHUNK

    for i in range(CHUNK // 16):
        idxg_v[pl.ds(16 * i, 16)] = (lane + gbase + 16 * i) >> 1
    gcp = pltpu.async_copy(x_hbm.at[idxg_v], xbuf_v, sem_g)

    pltpu.sync_copy(eidx_hbm.at[pl.ds(s * BLK, BLK)], eid_v)

    def h_body(i, carry):
        tot, half = carry
        v = eid_v[pl.ds(i * 16, 16)]
        contrib = z16
        def e_body(e, acc):
            m = v == e
            csum = plsc.cumsum(m.astype(jnp.int32))
            cnt = _take16(csum, jnp.full((16,), 15, jnp.int32))
            return acc + jnp.where(lane == e, cnt, 0)
        contrib = lax.fori_loop(0, E, e_body, contrib)
        flag = (i < 8).astype(jnp.int32)
        return tot + contrib, half + contrib * flag

    blk_hist, half_hist = lax.fori_loop(0, 16, h_body, (z16, z16))
    hstage_v[...] = blk_hist
    pltpu.sync_copy(hstage_v, hist_sh.at[pl.ds(s * 16, 16)])
    plsc.subcore_barrier()
    pltpu.sync_copy(hist_sh, hist_v)

    def r_body(jj, carry):
        tot, pre = carry
        row = hist_v[pl.ds(jj * 16, 16)]
        return tot + row, pre + row * (jj < s).astype(jnp.int32)

    totals, pre_blk = lax.fori_loop(0, NS, r_body, (z16, z16))

    tiles_e = (totals + (BG - 1)) >> 8
    ts_inc = plsc.cumsum(tiles_e)
    tile_start = ts_inc - tiles_e
    slot_base = tile_start * BG
    bases = slot_base + pre_blk + half_hist * c

    used = _take16(ts_inc, jnp.full((16,), E - 1, jnp.int32))
    for jv in range(2):
        jvec = lane + 16 * jv
        acc = jnp.full((16,), -1, jnp.int32)
        for e in range(E):
            st = _take16(tile_start, jnp.full((16,), e, jnp.int32))
            acc = acc + (jvec >= st).astype(jnp.int32)
        if jv == 1:
            acc = jnp.where(lane == 15, used, acc)
        te_v[pl.ds(16 * jv, 16)] = acc

    @pl.when(wid == 0)
    def _():
        pltpu.sync_copy(te_v, te_hbm)

    def j_body(j, run):
        v = eid_v[pl.ds(c * CHUNK + 16 * j, 16)]
        rank = z16
        cnt = z16
        for e in range(E):
            m = v == e
            csum = plsc.cumsum(m.astype(jnp.int32))
            rank = jnp.where(m, csum, rank)
            cnt = cnt + jnp.where(lane == e,
                                  _take16(csum, jnp.full((16,), 15, jnp.int32)), 0)
        basev = _take16(bases + run, v)
        pos16 = basev + rank - 1
        pos_v[pl.ds(16 * j, 16)] = pos16
        posw_v[0, pl.ds(16 * j, 16)] = pos16
        return run + cnt

    lax.fori_loop(0, CHUNK // 16, j_body, z16)
    gcp.wait()
    pltpu.async_copy(xbuf_v, xs_hbm.at[posw_v.at[0]], sem_s).wait()
    pltpu.sync_copy(pos_v, pos_hbm.at[pl.ds(gbase, CHUNK)])


def _dispatch(eidx_flat, x):
    mesh = plsc.VectorSubcoreMesh(core_axis_name="c", subcore_axis_name="s", num_cores=NC, num_subcores=NS)
    f = pl.kernel(
        _dispatch_body,
        out_type=[
            jax.ShapeDtypeStruct((SLOTS, D), jnp.float32),
            jax.ShapeDtypeStruct((TK,), jnp.int32),
            jax.ShapeDtypeStruct((32,), jnp.int32),
        ],
        mesh=mesh,
        compiler_params=pltpu.CompilerParams(needs_layout_passes=False),
        scratch_types=[
            pltpu.VMEM((BLK,), jnp.int32),
            pltpu.VMEM((16,), jnp.int32),
            pltpu.VMEM((NS * 16,), jnp.int32),
            pltpu.VMEM((CHUNK,), jnp.int32),
            pltpu.VMEM((1, CHUNK), jnp.int32),
            pltpu.VMEM((CHUNK,), jnp.int32),
            pltpu.VMEM((CHUNK, D), jnp.float32),
            pltpu.VMEM((32,), jnp.int32),
            pltpu.VMEM_SHARED((NS * 16,), jnp.int32),
            pltpu.SemaphoreType.DMA,
            pltpu.SemaphoreType.DMA,
        ],
    )
    return f(eidx_flat, x)


def _group_body(te_ref, xs_ref, w1_ref, w2_ref, ys_ref):
    @pl.when(pl.program_id(0) < te_ref[31])
    def _():
        x = xs_ref[...]
        h = jnp.dot(x, w1_ref[0], preferred_element_type=jnp.float32)
        a = _silu(h[:, :F]) * h[:, F:]
        ys_ref[...] = jnp.dot(a, w2_ref[0], preferred_element_type=jnp.float32)


def _grouped(tile_expert, x_slots, w1, w2):
    grid_spec = pltpu.PrefetchScalarGridSpec(
        num_scalar_prefetch=1,
        grid=(MAXT,),
        in_specs=[
            pl.BlockSpec((BG, D), lambda t, te: (t, 0)),
            pl.BlockSpec((1, D, 2 * F), lambda t, te: (te[t], 0, 0)),
            pl.BlockSpec((1, F, D), lambda t, te: (te[t], 0, 0)),
        ],
        out_specs=pl.BlockSpec((BG, D), lambda t, te: (t, 0)),
    )
    return pl.pallas_call(
        _group_body,
        grid_spec=grid_spec,
        out_shape=jax.ShapeDtypeStruct((SLOTS, D), jnp.float32),
    )(tile_expert, x_slots, w1, w2)


def _combine_body(ys_hbm, pos_hbm, wpair_hbm, routed_hbm, pos_v, wv_v,
                  posmat_v, ybuf_a, ybuf_b, obuf_v, sem_a, sem_b):
    c = lax.axis_index("c")
    s = lax.axis_index("s")
    wid = s * NC + c
    nj = TPW // 8
    pltpu.sync_copy(pos_hbm.at[pl.ds(wid * (2 * TPW), 2 * TPW)], pos_v)
    pltpu.sync_copy(wpair_hbm.at[pl.ds(wid * (2 * TPW), 2 * TPW)], wv_v)
    for j in range(nj):
        posmat_v[j, :] = pos_v[pl.ds(16 * j, 16)]

    bufs = (ybuf_a, ybuf_b)
    sems = (sem_a, sem_b)
    cps = [pltpu.async_copy(ys_hbm.at[posmat_v.at[0]], bufs[0], sems[0])]
    for j in range(nj):
        ybuf_v = bufs[j % 2]
        if j + 1 < nj:
            cps.append(pltpu.async_copy(ys_hbm.at[posmat_v.at[j + 1]],
                                        bufs[(j + 1) % 2], sems[(j + 1) % 2]))
        cps[j].wait()
        w16 = wv_v[pl.ds(16 * j, 16)]
        wbc = [_take16(w16, jnp.full((16,), r, jnp.int32)) for r in range(16)]

        def k_body(k, _2):
            for i in range(8):
                a = ybuf_v[2 * i, pl.ds(16 * k, 16)]
                b = ybuf_v[2 * i + 1, pl.ds(16 * k, 16)]
                obuf_v[i, pl.ds(16 * k, 16)] = a * wbc[2 * i] + b * wbc[2 * i + 1]
            return 0

        lax.fori_loop(0, D // 16, k_body, 0)
        pltpu.sync_copy(obuf_v, routed_hbm.at[pl.ds(wid * TPW + 8 * j, 8)])


def _combine(y_slots, pos_pair, wpair_flat):
    mesh = plsc.VectorSubcoreMesh(core_axis_name="c", subcore_axis_name="s", num_cores=NC, num_subcores=NS)
    f = pl.kernel(
        _combine_body,
        out_type=jax.ShapeDtypeStruct((T, D), jnp.float32),
        mesh=mesh,
        compiler_params=pltpu.CompilerParams(needs_layout_passes=False),
        scratch_types=[
            pltpu.VMEM((2 * TPW,), jnp.int32),
            pltpu.VMEM((2 * TPW,), jnp.float32),
            pltpu.VMEM((TPW // 8, 16), jnp.int32),
            pltpu.VMEM((16, D), jnp.float32),
            pltpu.VMEM((16, D), jnp.float32),
            pltpu.VMEM((8, D), jnp.float32),
            pltpu.SemaphoreType.DMA,
            pltpu.SemaphoreType.DMA,
        ],
    )
    return f(y_slots, pos_pair, wpair_flat)


def _shared_body(x_ref, ws1_ref, ws2_ref, out_ref):
    x = x_ref[...]
    h = jnp.dot(x, ws1_ref[...], preferred_element_type=jnp.float32)
    a = _silu(h[:, :F]) * h[:, F:]
    out_ref[...] = jnp.dot(a, ws2_ref[...], preferred_element_type=jnp.float32)


def _shared(x, ws1, ws2):
    return pl.pallas_call(
        _shared_body,
        grid=(NT,),
        in_specs=[
            pl.BlockSpec((BT, D), lambda t: (t, 0)),
            pl.BlockSpec((D, 2 * F), lambda t: (0, 0)),
            pl.BlockSpec((F, D), lambda t: (0, 0)),
        ],
        out_specs=pl.BlockSpec((BT, D), lambda t: (t, 0)),
        out_shape=jax.ShapeDtypeStruct((T, D), jnp.float32),
    )(x, ws1, ws2)


def kernel(hidden_states, gate_w, w1, w2, ws1, ws2):
    orig_shape = hidden_states.shape
    x = hidden_states.reshape(-1, D)
    eidx, wpair = _router(x, gate_w.T)
    x_slots, pos_pair, tile_expert = _dispatch(eidx.reshape(TK), x)
    shared = _shared(x, ws1, ws2)
    y_slots = _grouped(tile_expert, x_slots, w1, w2)
    routed = _combine(y_slots, pos_pair, wpair.reshape(TK))
    return shared, routed.reshape(orig_shape)
